# Initial kernel scaffold; baseline (speedup 1.0000x reference)
import jax
import jax.numpy as jnp
from jax.experimental import pallas as pl


def kernel(dense_input, sparse_input, embed_tables, wide_W, wide_b, W1, b1, W2, b2, W3, b3, Wout, bout):
    raise NotImplementedError("write your pallas kernel here")


# speedup vs baseline: 2.2016x; 2.2016x over previous
import jax
import jax.numpy as jnp
from jax import lax
from jax.experimental import pallas as pl
from jax.experimental.pallas import tpu as pltpu
from jax.experimental.pallas import tpu_sc as plsc

_NUM_FIELDS = 26
_VOCAB = 100000
_EMBED_DIM = 32
_BATCH = 4096
_DENSE = 13

_NC, _NS = 2, 16
_NW = _NC * _NS
_ROWS = _BATCH * _NUM_FIELDS
_RPW = _ROWS // _NW
_CHUNK = 128
_NCHUNK = _RPW // _CHUNK


def _gather_body(tbl_hbm, idx_hbm, out_hbm, idx_v, rows_v, sem):
    wid = lax.axis_index("s") * _NC + lax.axis_index("c")
    pltpu.sync_copy(idx_hbm.at[wid], idx_v)

    for j in range(_NCHUNK):
        for k in range(_CHUNK // 16):
            s = pl.ds(k * 16, 16)
            p = j * _CHUNK + k * 16 + lax.iota(jnp.int32, 16)
            off = lax.rem(p, _NUM_FIELDS) * _VOCAB
            idx_v[j, s] = idx_v[j, s] + off

    copies = [
        pltpu.async_copy(
            tbl_hbm.at[idx_v.at[j]],
            rows_v.at[pl.ds(j * _CHUNK, _CHUNK)],
            sem,
        )
        for j in range(_NCHUNK)
    ]
    for c in copies:
        c.wait()

    pltpu.sync_copy(rows_v, out_hbm.at[pl.ds(wid * _RPW, _RPW)])


_gather_cache = []


def _gather_kernel():
    if not _gather_cache:
        mesh = plsc.VectorSubcoreMesh(
            core_axis_name="c", subcore_axis_name="s",
            num_cores=_NC, num_subcores=_NS,
        )
        _gather_cache.append(pl.kernel(
            _gather_body,
            out_type=jax.ShapeDtypeStruct((_ROWS, _EMBED_DIM), jnp.float32),
            mesh=mesh,
            scratch_types=[
                pltpu.VMEM((_NCHUNK, _CHUNK), jnp.int32),
                pltpu.VMEM((_RPW, _EMBED_DIM), jnp.float32),
                pltpu.SemaphoreType.DMA,
            ],
            compiler_params=pltpu.CompilerParams(use_tc_tiling_on_sc=False),
        ))
    return _gather_cache[0]


_BB = 512


def _mlp_body(emb_ref, dense_ref, w1_ref, b1_ref, w2_ref, b2_ref, w3_ref,
              b3_ref, wout_ref, wide_w_ref, wide_b_ref, bout_ref, out_ref):
    dn = (((1,), (1,)), ((), ()))
    f32 = jnp.float32
    x = emb_ref[...]
    h = jnp.maximum(
        lax.dot_general(x, w1_ref[...], dn, preferred_element_type=f32)
        + b1_ref[...][None, :], 0.0)
    h = jnp.maximum(
        lax.dot_general(h, w2_ref[...], dn, preferred_element_type=f32)
        + b2_ref[...][None, :], 0.0)
    h = jnp.maximum(
        lax.dot_general(h, w3_ref[...], dn, preferred_element_type=f32)
        + b3_ref[...][None, :], 0.0)
    deep = jnp.sum(h * wout_ref[...], axis=1) + bout_ref[0]
    wide = jnp.sum(dense_ref[...] * wide_w_ref[...], axis=1) + wide_b_ref[0]
    out_ref[...] = jax.nn.sigmoid(0.5 * (wide + deep))


def _mlp(emb, dense_input, W1, b1, W2, b2, W3, b3, Wout, wide_W, wide_b, bout):
    rep2 = lambda i: (0, 0)
    rep1 = lambda i: (0,)
    return pl.pallas_call(
        _mlp_body,
        grid=(_BATCH // _BB,),
        in_specs=[
            pl.BlockSpec((_BB, _NUM_FIELDS * _EMBED_DIM), lambda i: (i, 0)),
            pl.BlockSpec((_BB, _DENSE), lambda i: (i, 0)),
            pl.BlockSpec(W1.shape, rep2),
            pl.BlockSpec(b1.shape, rep1),
            pl.BlockSpec(W2.shape, rep2),
            pl.BlockSpec(b2.shape, rep1),
            pl.BlockSpec(W3.shape, rep2),
            pl.BlockSpec(b3.shape, rep1),
            pl.BlockSpec(Wout.shape, rep2),
            pl.BlockSpec(wide_W.shape, rep2),
            pl.BlockSpec(memory_space=pltpu.SMEM),
            pl.BlockSpec(memory_space=pltpu.SMEM),
        ],
        out_specs=pl.BlockSpec((_BB,), lambda i: (i,)),
        out_shape=jax.ShapeDtypeStruct((_BATCH,), jnp.float32),
    )(emb, dense_input, W1, b1, W2, b2, W3, b3, Wout, wide_W, wide_b, bout)


def kernel(dense_input, sparse_input, embed_tables, wide_W, wide_b,
           W1, b1, W2, b2, W3, b3, Wout, bout):
    idx = sparse_input.astype(jnp.int32).reshape(_NW, _NCHUNK, _CHUNK)
    tbl = embed_tables.reshape(_NUM_FIELDS * _VOCAB, _EMBED_DIM)
    rows = _gather_kernel()(tbl, idx)
    emb = rows.reshape(_BATCH, _NUM_FIELDS * _EMBED_DIM)
    return _mlp(emb, dense_input, W1, b1, W2, b2, W3, b3, Wout,
                wide_W, wide_b, bout)


# speedup vs baseline: 11.6063x; 5.2716x over previous
import jax
import jax.numpy as jnp
from jax import lax
from jax.experimental import pallas as pl
from jax.experimental.pallas import tpu as pltpu
from jax.experimental.pallas import tpu_sc as plsc

_NUM_FIELDS = 26
_VOCAB = 100000
_EMBED_DIM = 32
_BATCH = 4096
_DENSE = 13
_FD = _NUM_FIELDS * _EMBED_DIM

_NC, _NS = 2, 16
_NW = _NC * _NS


def _gather_body(tblT_hbm, idxT_hbm, out_hbm, row_v, idx_v, out_v):
    wid = lax.axis_index("s") * _NC + lax.axis_index("c")
    for f in range(_NUM_FIELDS):
        r = f * _EMBED_DIM + wid
        pltpu.sync_copy(tblT_hbm.at[r], row_v)
        pltpu.sync_copy(idxT_hbm.at[f], idx_v)

        def _chunk(t, c):
            s = pl.ds(t * 16, 16)
            out_v[s] = plsc.load_gather(row_v, [idx_v[s]])
            return c
        lax.fori_loop(0, _BATCH // 16, _chunk, 0)
        pltpu.sync_copy(out_v, out_hbm.at[r])


_gather_cache = []


def _gather_kernel():
    if not _gather_cache:
        mesh = plsc.VectorSubcoreMesh(
            core_axis_name="c", subcore_axis_name="s",
            num_cores=_NC, num_subcores=_NS,
        )
        _gather_cache.append(pl.kernel(
            _gather_body,
            out_type=jax.ShapeDtypeStruct((_FD, _BATCH), jnp.float32),
            mesh=mesh,
            scratch_types=[
                pltpu.VMEM((_VOCAB,), jnp.float32),
                pltpu.VMEM((_BATCH,), jnp.int32),
                pltpu.VMEM((_BATCH,), jnp.float32),
            ],
            compiler_params=pltpu.CompilerParams(needs_layout_passes=False),
        ))
    return _gather_cache[0]


_BB = 512


def _mlp_body(embT_ref, dense_ref, w1_ref, b1_ref, w2_ref, b2_ref, w3_ref,
              b3_ref, wout_ref, wide_w_ref, wide_b_ref, bout_ref, out_ref):
    dn = (((1,), (1,)), ((), ()))
    f32 = jnp.float32
    x = embT_ref[...]
    h = jnp.maximum(
        lax.dot_general(x, w1_ref[...], (((0,), (1,)), ((), ())),
                        preferred_element_type=f32)
        + b1_ref[...][None, :], 0.0)
    h = jnp.maximum(
        lax.dot_general(h, w2_ref[...], dn, preferred_element_type=f32)
        + b2_ref[...][None, :], 0.0)
    h = jnp.maximum(
        lax.dot_general(h, w3_ref[...], dn, preferred_element_type=f32)
        + b3_ref[...][None, :], 0.0)
    deep = jnp.sum(h * wout_ref[...], axis=1) + bout_ref[0]
    wide = jnp.sum(dense_ref[...] * wide_w_ref[...], axis=1) + wide_b_ref[0]
    out_ref[...] = jax.nn.sigmoid(0.5 * (wide + deep))


def _mlp(embT, dense_input, W1, b1, W2, b2, W3, b3, Wout, wide_W, wide_b,
         bout):
    rep2 = lambda i: (0, 0)
    rep1 = lambda i: (0,)
    return pl.pallas_call(
        _mlp_body,
        grid=(_BATCH // _BB,),
        in_specs=[
            pl.BlockSpec((_FD, _BB), lambda i: (0, i)),
            pl.BlockSpec((_BB, _DENSE), lambda i: (i, 0)),
            pl.BlockSpec(W1.shape, rep2),
            pl.BlockSpec(b1.shape, rep1),
            pl.BlockSpec(W2.shape, rep2),
            pl.BlockSpec(b2.shape, rep1),
            pl.BlockSpec(W3.shape, rep2),
            pl.BlockSpec(b3.shape, rep1),
            pl.BlockSpec(Wout.shape, rep2),
            pl.BlockSpec(wide_W.shape, rep2),
            pl.BlockSpec(memory_space=pltpu.SMEM),
            pl.BlockSpec(memory_space=pltpu.SMEM),
        ],
        out_specs=pl.BlockSpec((_BB,), lambda i: (i,)),
        out_shape=jax.ShapeDtypeStruct((_BATCH,), jnp.float32),
    )(embT, dense_input, W1, b1, W2, b2, W3, b3, Wout, wide_W, wide_b, bout)


def kernel(dense_input, sparse_input, embed_tables, wide_W, wide_b,
           W1, b1, W2, b2, W3, b3, Wout, bout):
    tblT = embed_tables.transpose(0, 2, 1).reshape(_FD, _VOCAB)
    idxT = sparse_input.astype(jnp.int32).T
    embT = _gather_kernel()(tblT, idxT)
    return _mlp(embT, dense_input, W1, b1, W2, b2, W3, b3, Wout,
                wide_W, wide_b, bout)


# speedup vs baseline: 13.6760x; 1.1783x over previous
import jax
import jax.numpy as jnp
from jax import lax
from jax.experimental import pallas as pl
from jax.experimental.pallas import tpu as pltpu
from jax.experimental.pallas import tpu_sc as plsc

_NUM_FIELDS = 26
_VOCAB = 100000
_EMBED_DIM = 32
_BATCH = 4096
_DENSE = 13
_FD = _NUM_FIELDS * _EMBED_DIM

_NC, _NS = 2, 16
_NW = _NC * _NS


_H0 = 51200
_H1 = _VOCAB - _H0
_HALVES = ((0, _H0), (_H0, _H1))
_NSTEP = 2 * _NUM_FIELDS


def _gather_body(tblT_hbm, idxT_hbm, tail_hbm, out_hbm, rowA, rowB, idx_v,
                 out_v, semA, semB):
    wid = lax.axis_index("s") * _NC + lax.axis_index("c")
    bufs = (rowA, rowB)
    sems = (semA, semB)

    def _start(step):
        f, h = divmod(step, 2)
        lo, n = _HALVES[h]
        r = f * _EMBED_DIM + wid
        if h == 0:
            return [pltpu.async_copy(
                tblT_hbm.at[r, pl.ds(lo, n)],
                bufs[step % 2].at[pl.ds(0, n)],
                sems[step % 2],
            )]
        n0 = n - 32
        return [
            pltpu.async_copy(
                tblT_hbm.at[r, pl.ds(lo, n0)],
                bufs[step % 2].at[pl.ds(0, n0)],
                sems[step % 2],
            ),
            pltpu.async_copy(
                tail_hbm.at[r],
                bufs[step % 2].at[pl.ds(n0, 128)],
                sems[step % 2],
            ),
        ]

    cp = _start(0)
    for step in range(_NSTEP):
        f, h = divmod(step, 2)
        nxt = _start(step + 1) if step + 1 < _NSTEP else None
        if h == 0:
            pltpu.sync_copy(idxT_hbm.at[f], idx_v)
        for c in cp:
            c.wait()
        buf = bufs[step % 2]

        def _chunk(t, c, buf=buf, h=h):
            s = pl.ds(t * 16, 16)
            i16 = idx_v[s]
            if h == 0:
                m = i16 < _H0
                g = plsc.load_gather(buf, [i16], mask=m)
                out_v[s] = g
            else:
                adj = i16 - _H0
                m = adj >= 0
                g = plsc.load_gather(buf, [adj], mask=m)
                out_v[s] = jnp.where(m, g, out_v[s])
            return c
        lax.fori_loop(0, _BATCH // 16, _chunk, 0)
        if h == 1:
            pltpu.sync_copy(out_v, out_hbm.at[f * _EMBED_DIM + wid])
        cp = nxt


_gather_cache = []


def _gather_kernel():
    if not _gather_cache:
        mesh = plsc.VectorSubcoreMesh(
            core_axis_name="c", subcore_axis_name="s",
            num_cores=_NC, num_subcores=_NS,
        )
        _gather_cache.append(pl.kernel(
            _gather_body,
            out_type=jax.ShapeDtypeStruct((_FD, _BATCH), jnp.float32),
            mesh=mesh,
            scratch_types=[
                pltpu.VMEM((_H0,), jnp.float32),
                pltpu.VMEM((_H0,), jnp.float32),
                pltpu.VMEM((_BATCH,), jnp.int32),
                pltpu.VMEM((_BATCH,), jnp.float32),
                pltpu.SemaphoreType.DMA,
                pltpu.SemaphoreType.DMA,
            ],
            compiler_params=pltpu.CompilerParams(needs_layout_passes=False),
        ))
    return _gather_cache[0]


_BB = 512


def _mlp_body(embT_ref, dense_ref, w1_ref, b1_ref, w2_ref, b2_ref, w3_ref,
              b3_ref, wout_ref, wide_w_ref, wide_b_ref, bout_ref, out_ref):
    dn = (((1,), (1,)), ((), ()))
    f32 = jnp.float32
    x = embT_ref[...]
    h = jnp.maximum(
        lax.dot_general(x, w1_ref[...], (((0,), (1,)), ((), ())),
                        preferred_element_type=f32)
        + b1_ref[...][None, :], 0.0)
    h = jnp.maximum(
        lax.dot_general(h, w2_ref[...], dn, preferred_element_type=f32)
        + b2_ref[...][None, :], 0.0)
    h = jnp.maximum(
        lax.dot_general(h, w3_ref[...], dn, preferred_element_type=f32)
        + b3_ref[...][None, :], 0.0)
    deep = jnp.sum(h * wout_ref[...], axis=1) + bout_ref[0]
    wide = jnp.sum(dense_ref[...] * wide_w_ref[...], axis=1) + wide_b_ref[0]
    out_ref[...] = jax.nn.sigmoid(0.5 * (wide + deep))


def _mlp(embT, dense_input, W1, b1, W2, b2, W3, b3, Wout, wide_W, wide_b,
         bout):
    rep2 = lambda i: (0, 0)
    rep1 = lambda i: (0,)
    return pl.pallas_call(
        _mlp_body,
        grid=(_BATCH // _BB,),
        in_specs=[
            pl.BlockSpec((_FD, _BB), lambda i: (0, i)),
            pl.BlockSpec((_BB, _DENSE), lambda i: (i, 0)),
            pl.BlockSpec(W1.shape, rep2),
            pl.BlockSpec(b1.shape, rep1),
            pl.BlockSpec(W2.shape, rep2),
            pl.BlockSpec(b2.shape, rep1),
            pl.BlockSpec(W3.shape, rep2),
            pl.BlockSpec(b3.shape, rep1),
            pl.BlockSpec(Wout.shape, rep2),
            pl.BlockSpec(wide_W.shape, rep2),
            pl.BlockSpec(memory_space=pltpu.SMEM),
            pl.BlockSpec(memory_space=pltpu.SMEM),
        ],
        out_specs=pl.BlockSpec((_BB,), lambda i: (i,)),
        out_shape=jax.ShapeDtypeStruct((_BATCH,), jnp.float32),
    )(embT, dense_input, W1, b1, W2, b2, W3, b3, Wout, wide_W, wide_b, bout)


def kernel(dense_input, sparse_input, embed_tables, wide_W, wide_b,
           W1, b1, W2, b2, W3, b3, Wout, bout):
    tblT = embed_tables.transpose(0, 2, 1).reshape(_FD, _VOCAB)
    idxT = sparse_input.astype(jnp.int32).T
    tail = jnp.pad(tblT[:, _VOCAB - 32:], ((0, 0), (0, 96)))
    embT = _gather_kernel()(tblT, idxT, tail)
    return _mlp(embT, dense_input, W1, b1, W2, b2, W3, b3, Wout,
                wide_W, wide_b, bout)


# speedup vs baseline: 15.1060x; 1.1046x over previous
import jax
import jax.numpy as jnp
from jax import lax
from jax.experimental import pallas as pl
from jax.experimental.pallas import tpu as pltpu
from jax.experimental.pallas import tpu_sc as plsc

_NUM_FIELDS = 26
_VOCAB = 100000
_EMBED_DIM = 32
_BATCH = 4096
_DENSE = 13
_FD = _NUM_FIELDS * _EMBED_DIM

_NC, _NS = 2, 16
_NW = _NC * _NS


_H0 = 51200
_H1 = _VOCAB - _H0
_HALVES = ((0, _H0), (_H0, _H1))
_NSTEP = 2 * _NUM_FIELDS


def _gather_body(tblT_hbm, idxT_hbm, tail_hbm, out_hbm, rowA, rowB, idx_v,
                 out_v, semA, semB):
    wid = lax.axis_index("s") * _NC + lax.axis_index("c")
    bufs = (rowA, rowB)
    sems = (semA, semB)

    def _start(step):
        f, h = divmod(step, 2)
        lo, n = _HALVES[h]
        r = f * _EMBED_DIM + wid
        if h == 0:
            return [pltpu.async_copy(
                tblT_hbm.at[r, pl.ds(lo, n)],
                bufs[step % 2].at[pl.ds(0, n)],
                sems[step % 2],
            )]
        n0 = n - 32
        return [
            pltpu.async_copy(
                tblT_hbm.at[r, pl.ds(lo, n0)],
                bufs[step % 2].at[pl.ds(0, n0)],
                sems[step % 2],
            ),
            pltpu.async_copy(
                tail_hbm.at[r],
                bufs[step % 2].at[pl.ds(n0, 128)],
                sems[step % 2],
            ),
        ]

    cp = _start(0)
    for step in range(_NSTEP):
        f, h = divmod(step, 2)
        nxt = _start(step + 1) if step + 1 < _NSTEP else None
        if h == 0:
            pltpu.sync_copy(idxT_hbm.at[f], idx_v)
        for c in cp:
            c.wait()
        buf = bufs[step % 2]

        @plsc.parallel_loop(0, _BATCH, 16, unroll=4)
        def _chunk(i, buf=buf, h=h):
            s = pl.ds(i, 16)
            i16 = idx_v[s]
            if h == 0:
                m = i16 < _H0
                g = plsc.load_gather(buf, [i16], mask=m)
                out_v[s] = g
            else:
                adj = i16 - _H0
                m = adj >= 0
                g = plsc.load_gather(buf, [adj], mask=m)
                out_v[s] = jnp.where(m, g, out_v[s])
        if h == 1:
            pltpu.sync_copy(out_v, out_hbm.at[f * _EMBED_DIM + wid])
        cp = nxt


_gather_cache = []


def _gather_kernel():
    if not _gather_cache:
        mesh = plsc.VectorSubcoreMesh(
            core_axis_name="c", subcore_axis_name="s",
            num_cores=_NC, num_subcores=_NS,
        )
        _gather_cache.append(pl.kernel(
            _gather_body,
            out_type=jax.ShapeDtypeStruct((_FD, _BATCH), jnp.float32),
            mesh=mesh,
            scratch_types=[
                pltpu.VMEM((_H0,), jnp.float32),
                pltpu.VMEM((_H0,), jnp.float32),
                pltpu.VMEM((_BATCH,), jnp.int32),
                pltpu.VMEM((_BATCH,), jnp.float32),
                pltpu.SemaphoreType.DMA,
                pltpu.SemaphoreType.DMA,
            ],
            compiler_params=pltpu.CompilerParams(needs_layout_passes=False),
        ))
    return _gather_cache[0]


_BB = 512


def _mlp_body(embT_ref, dense_ref, w1_ref, b1_ref, w2_ref, b2_ref, w3_ref,
              b3_ref, wout_ref, wide_w_ref, wide_b_ref, bout_ref, out_ref):
    dn = (((1,), (1,)), ((), ()))
    f32 = jnp.float32
    x = embT_ref[...]
    h = jnp.maximum(
        lax.dot_general(x, w1_ref[...], (((0,), (1,)), ((), ())),
                        preferred_element_type=f32)
        + b1_ref[...][None, :], 0.0)
    h = jnp.maximum(
        lax.dot_general(h, w2_ref[...], dn, preferred_element_type=f32)
        + b2_ref[...][None, :], 0.0)
    h = jnp.maximum(
        lax.dot_general(h, w3_ref[...], dn, preferred_element_type=f32)
        + b3_ref[...][None, :], 0.0)
    deep = jnp.sum(h * wout_ref[...], axis=1) + bout_ref[0]
    wide = jnp.sum(dense_ref[...] * wide_w_ref[...], axis=1) + wide_b_ref[0]
    out_ref[...] = jax.nn.sigmoid(0.5 * (wide + deep))


def _mlp(embT, dense_input, W1, b1, W2, b2, W3, b3, Wout, wide_W, wide_b,
         bout):
    rep2 = lambda i: (0, 0)
    rep1 = lambda i: (0,)
    return pl.pallas_call(
        _mlp_body,
        grid=(_BATCH // _BB,),
        in_specs=[
            pl.BlockSpec((_FD, _BB), lambda i: (0, i)),
            pl.BlockSpec((_BB, _DENSE), lambda i: (i, 0)),
            pl.BlockSpec(W1.shape, rep2),
            pl.BlockSpec(b1.shape, rep1),
            pl.BlockSpec(W2.shape, rep2),
            pl.BlockSpec(b2.shape, rep1),
            pl.BlockSpec(W3.shape, rep2),
            pl.BlockSpec(b3.shape, rep1),
            pl.BlockSpec(Wout.shape, rep2),
            pl.BlockSpec(wide_W.shape, rep2),
            pl.BlockSpec(memory_space=pltpu.SMEM),
            pl.BlockSpec(memory_space=pltpu.SMEM),
        ],
        out_specs=pl.BlockSpec((_BB,), lambda i: (i,)),
        out_shape=jax.ShapeDtypeStruct((_BATCH,), jnp.float32),
    )(embT, dense_input, W1, b1, W2, b2, W3, b3, Wout, wide_W, wide_b, bout)


def kernel(dense_input, sparse_input, embed_tables, wide_W, wide_b,
           W1, b1, W2, b2, W3, b3, Wout, bout):
    tblT = embed_tables.transpose(0, 2, 1).reshape(_FD, _VOCAB)
    idxT = sparse_input.astype(jnp.int32).T
    tail = jnp.pad(tblT[:, _VOCAB - 32:], ((0, 0), (0, 96)))
    embT = _gather_kernel()(tblT, idxT, tail)
    return _mlp(embT, dense_input, W1, b1, W2, b2, W3, b3, Wout,
                wide_W, wide_b, bout)
